# Initial kernel scaffold; baseline (speedup 1.0000x reference)
import functools

import jax
import jax.numpy as jnp
from jax import lax
from jax.experimental import pallas as pl
from jax.experimental.pallas import tpu as pltpu
from jax.experimental.pallas import tpu_sc as plsc

NUM_WORDS = 1000000
EMB = 32
TOTAL = 16384 * 50
NW = 32
PER_W = TOTAL // NW
SEG = 128
ROWS_PER_W = PER_W // SEG
K = 8
GROUPS = ROWS_PER_W // K


def _gather_body(idx_hbm, table_hbm, out_hbm, idx_v, rows_v, sem):
    wid = lax.axis_index("s") * 2 + lax.axis_index("c")
    base = wid * PER_W
    pltpu.sync_copy(idx_hbm.at[wid], idx_v)

    def group(g, carry):
        copies = []
        for j in range(K):
            copies.append(
                pltpu.async_copy(
                    table_hbm.at[idx_v.at[g * K + j]],
                    rows_v.at[pl.ds(j * SEG, SEG)],
                    sem,
                )
            )
        for c in copies:
            c.wait()
        pltpu.sync_copy(rows_v, out_hbm.at[pl.ds(base + g * (K * SEG), K * SEG)])
        return carry

    lax.fori_loop(0, GROUPS, group, 0)


@jax.jit
def kernel(indices, embedding_weight):
    idx = indices.reshape(TOTAL).astype(jnp.int32).reshape(NW, ROWS_PER_W, SEG)
    mesh = plsc.VectorSubcoreMesh(core_axis_name="c", subcore_axis_name="s")
    out = pl.kernel(
        _gather_body,
        mesh=mesh,
        out_type=jax.ShapeDtypeStruct((TOTAL, EMB), jnp.float32),
        scratch_types=[
            pltpu.VMEM((ROWS_PER_W, SEG), jnp.int32),
            pltpu.VMEM((K * SEG, EMB), jnp.float32),
            pltpu.SemaphoreType.DMA,
        ],
    )(idx, embedding_weight)
    return out.reshape(16384, 50, EMB)


# speedup vs baseline: 1.1025x; 1.1025x over previous
import functools

import jax
import jax.numpy as jnp
from jax import lax
from jax.experimental import pallas as pl
from jax.experimental.pallas import tpu as pltpu
from jax.experimental.pallas import tpu_sc as plsc

NUM_WORDS = 1000000
EMB = 32
TOTAL = 16384 * 50
NW = 32
PER_W = TOTAL // NW
SEG = 128
ROWS_PER_W = PER_W // SEG
K = 8
GROUPS = ROWS_PER_W // K


def _gather_body(idx_hbm, table_hbm, out_hbm, idx_v, rows_v, sem):
    wid = lax.axis_index("s") * 2 + lax.axis_index("c")
    base = wid * PER_W
    pltpu.sync_copy(idx_hbm.at[wid], idx_v)

    def group(g, carry):
        copies = []
        for j in range(K):
            copies.append(
                pltpu.async_copy(
                    table_hbm.at[idx_v.at[g * K + j]],
                    rows_v.at[pl.ds(j * SEG, SEG)],
                    sem,
                )
            )
        for c in copies:
            c.wait()
        pltpu.sync_copy(rows_v, out_hbm.at[pl.ds(base + g * (K * SEG), K * SEG)])
        return carry

    lax.fori_loop(0, GROUPS, group, 0)


@jax.jit
def kernel(indices, embedding_weight):
    idx = indices.reshape(TOTAL).astype(jnp.int32).reshape(NW, ROWS_PER_W, SEG)
    mesh = plsc.VectorSubcoreMesh(core_axis_name="c", subcore_axis_name="s")
    out = pl.kernel(
        _gather_body,
        mesh=mesh,
        out_type=jax.ShapeDtypeStruct((TOTAL, EMB), jnp.float32),
        compiler_params=pltpu.CompilerParams(use_tc_tiling_on_sc=False),
        scratch_types=[
            pltpu.VMEM((ROWS_PER_W, SEG), jnp.int32),
            pltpu.VMEM((K * SEG, EMB), jnp.float32),
            pltpu.SemaphoreType.DMA,
        ],
    )(idx, embedding_weight)
    return out.reshape(16384, 50, EMB)


# speedup vs baseline: 1.1104x; 1.0071x over previous
import functools

import jax
import jax.numpy as jnp
from jax import lax
from jax.experimental import pallas as pl
from jax.experimental.pallas import tpu as pltpu
from jax.experimental.pallas import tpu_sc as plsc

NUM_WORDS = 1000000
EMB = 32
TOTAL = 16384 * 50
NW = 32
PER_W = TOTAL // NW
SEG = 128
ROWS_PER_W = PER_W // SEG
S = 10
GROUP_ROWS = S * SEG
ITERS = ROWS_PER_W // (2 * S)


def _gather_body(idx_hbm, table_hbm, out_hbm, idx_v, buf_a, buf_b,
                 g_sem, wb_a, wb_b):
    wid = lax.axis_index("s") * 2 + lax.axis_index("c")
    base = wid * PER_W
    pltpu.sync_copy(idx_hbm.at[wid], idx_v)

    bufs = (buf_a, buf_b)
    wb_sems = (wb_a, wb_b)

    def step(i, carry):
        for half in range(2):
            buf, wb = bufs[half], wb_sems[half]
            g = 2 * i + half

            @pl.when(i >= 1)
            def _():
                pltpu.make_async_copy(
                    table_hbm.at[pl.ds(0, GROUP_ROWS)], buf, wb).wait()

            gathers = []
            for j in range(S):
                gathers.append(
                    pltpu.async_copy(
                        table_hbm.at[idx_v.at[g * S + j]],
                        buf.at[pl.ds(j * SEG, SEG)],
                        g_sem,
                    )
                )
            for c in gathers:
                c.wait()
            pltpu.async_copy(
                buf, out_hbm.at[pl.ds(base + g * GROUP_ROWS, GROUP_ROWS)], wb)
        return carry

    lax.fori_loop(0, ITERS, step, 0)

    for half in range(2):
        pltpu.make_async_copy(
            table_hbm.at[pl.ds(0, GROUP_ROWS)], bufs[half], wb_sems[half]).wait()


@jax.jit
def kernel(indices, embedding_weight):
    idx = indices.reshape(TOTAL).astype(jnp.int32).reshape(NW, ROWS_PER_W, SEG)
    mesh = plsc.VectorSubcoreMesh(core_axis_name="c", subcore_axis_name="s")
    out = pl.kernel(
        _gather_body,
        mesh=mesh,
        out_type=jax.ShapeDtypeStruct((TOTAL, EMB), jnp.float32),
        compiler_params=pltpu.CompilerParams(use_tc_tiling_on_sc=False),
        scratch_types=[
            pltpu.VMEM((ROWS_PER_W, SEG), jnp.int32),
            pltpu.VMEM((GROUP_ROWS, EMB), jnp.float32),
            pltpu.VMEM((GROUP_ROWS, EMB), jnp.float32),
            pltpu.SemaphoreType.DMA,
            pltpu.SemaphoreType.DMA,
            pltpu.SemaphoreType.DMA,
        ],
    )(idx, embedding_weight)
    return out.reshape(16384, 50, EMB)


# speedup vs baseline: 1.7920x; 1.6138x over previous
import functools

import jax
import jax.numpy as jnp
from jax import lax
from jax.experimental import pallas as pl
from jax.experimental.pallas import tpu as pltpu
from jax.experimental.pallas import tpu_sc as plsc

NUM_WORDS = 1000000
EMB = 32
BATCH = 16384
HIST = 50
NW = 32
ELEMS_PER_W = BATCH // NW
GROUP = 16
ITERS = ELEMS_PER_W // (2 * GROUP)


def _gather_body(idx_hbm, table_hbm, out_hbm, idx_v, buf_a, buf_b,
                 g_sem, wb_a, wb_b):
    wid = lax.axis_index("s") * 2 + lax.axis_index("c")
    base = wid * ELEMS_PER_W
    pltpu.sync_copy(idx_hbm.at[wid], idx_v)

    bufs = (buf_a, buf_b)
    wb_sems = (wb_a, wb_b)

    def step(i, carry):
        for half in range(2):
            buf, wb = bufs[half], wb_sems[half]
            g = 2 * i + half

            @pl.when(i >= 1)
            def _():
                pltpu.make_async_copy(
                    out_hbm.at[pl.ds(0, GROUP)], buf, wb).wait()

            gathers = []
            for j in range(GROUP):
                gathers.append(
                    pltpu.async_copy(
                        table_hbm.at[idx_v.at[g * GROUP + j]],
                        buf.at[j],
                        g_sem,
                    )
                )
            for c in gathers:
                c.wait()
            pltpu.async_copy(
                buf, out_hbm.at[pl.ds(base + g * GROUP, GROUP)], wb)
        return carry

    lax.fori_loop(0, ITERS, step, 0)

    for half in range(2):
        pltpu.make_async_copy(
            out_hbm.at[pl.ds(0, GROUP)], bufs[half], wb_sems[half]).wait()


@jax.jit
def kernel(indices, embedding_weight):
    idx = indices.astype(jnp.int32).reshape(NW, ELEMS_PER_W, HIST)
    mesh = plsc.VectorSubcoreMesh(core_axis_name="c", subcore_axis_name="s")
    out = pl.kernel(
        _gather_body,
        mesh=mesh,
        out_type=jax.ShapeDtypeStruct((BATCH, HIST, EMB), jnp.float32),
        compiler_params=pltpu.CompilerParams(use_tc_tiling_on_sc=False),
        scratch_types=[
            pltpu.VMEM((ELEMS_PER_W, HIST), jnp.int32),
            pltpu.VMEM((GROUP, HIST, EMB), jnp.float32),
            pltpu.VMEM((GROUP, HIST, EMB), jnp.float32),
            pltpu.SemaphoreType.DMA,
            pltpu.SemaphoreType.DMA,
            pltpu.SemaphoreType.DMA,
        ],
    )(idx, embedding_weight)
    return out
